# baseline (speedup 1.0000x reference)
import functools

import jax
import jax.numpy as jnp
from jax import lax
from jax.experimental import pallas as pl
from jax.experimental.pallas import tpu as pltpu
from jax.experimental.pallas import tpu_sc as plsc

BLK = 32


def _sc_embed(x_flat, mask_i32, bos_emb, eos_emb, zeros, B, T, C):
    To = T + 2
    NC, NS = 2, 16
    NW = NC * NS
    WPB = NW // B
    H = To // WPB
    assert WPB * H == To

    mesh = plsc.VectorSubcoreMesh(core_axis_name="c", subcore_axis_name="s")

    @functools.partial(
        pl.kernel,
        mesh=mesh,
        compiler_params=pltpu.CompilerParams(needs_layout_passes=False),
        out_type=[
            jax.ShapeDtypeStruct((B * To * C,), jnp.float32),
            jax.ShapeDtypeStruct((B,), jnp.int32),
        ],
        scratch_types=[
            pltpu.VMEM((T,), jnp.int32),
            pltpu.VMEM((BLK * C,), jnp.float32),
            pltpu.VMEM((BLK * C,), jnp.float32),
            pltpu.VMEM((C,), jnp.float32),
            pltpu.VMEM((C,), jnp.float32),
            pltpu.VMEM((C,), jnp.float32),
            pltpu.VMEM((B,), jnp.int32),
        ],
    )
    def body(x_hbm, m_hbm, bos_hbm, eos_hbm, z_hbm, out_hbm, len_hbm,
             mbuf, buf, zbuf, rowbuf, bosbuf, eosbuf, lenbuf):
        c = lax.axis_index("c")
        s = lax.axis_index("s")
        wid = s * NC + c
        b = wid // WPB
        h = wid % WPB
        s0 = h * H
        s1 = s0 + H
        ob = b * (To * C)
        xb = b * (T * C)

        pltpu.sync_copy(bos_hbm, bosbuf)
        pltpu.sync_copy(eos_hbm, eosbuf)
        pltpu.sync_copy(z_hbm, zbuf)

        def row_len(bb):
            pltpu.sync_copy(m_hbm.at[pl.ds(bb * T, T)], mbuf)

            def sbody(i, acc):
                return acc + mbuf[pl.ds(i * 16, 16)]

            acc = lax.fori_loop(0, T // 16, sbody, jnp.zeros((16,), jnp.int32))
            return T - jnp.sum(acc)

        len0 = row_len(b)
        E = len0 + 1

        zs = jnp.maximum(s0, len0 + 2)
        Z = jnp.maximum(s1 - zs, 0)
        remh = Z % BLK

        def zrow(k, carry):
            pltpu.sync_copy(zbuf.at[pl.ds(0, C)],
                            out_hbm.at[pl.ds(ob + (zs + k) * C, C)])
            return carry

        lax.fori_loop(0, remh, zrow, 0)

        def zblk(i, carry):
            pltpu.sync_copy(
                zbuf, out_hbm.at[pl.ds(ob + (zs + remh + i * BLK) * C, BLK * C)])
            return carry

        lax.fori_loop(0, Z // BLK, zblk, 0)

        cs = jnp.maximum(s0, 1)
        ce = jnp.minimum(s1, E)
        L = jnp.maximum(ce - cs, 0)
        nfull = L // BLK

        def cblk(i, carry):
            t = cs + i * BLK
            pltpu.sync_copy(x_hbm.at[pl.ds(xb + (t - 1) * C, BLK * C)], buf)
            pltpu.sync_copy(buf, out_hbm.at[pl.ds(ob + t * C, BLK * C)])
            return carry

        lax.fori_loop(0, nfull, cblk, 0)

        rem = L - nfull * BLK
        base = cs + nfull * BLK

        def crow(k, carry):
            pltpu.sync_copy(x_hbm.at[pl.ds(xb + (base - 1 + k) * C, C)], rowbuf)
            pltpu.sync_copy(rowbuf, out_hbm.at[pl.ds(ob + (base + k) * C, C)])
            return carry

        lax.fori_loop(0, rem, crow, 0)

        @pl.when((E >= s0) & (E < s1))
        def _():
            pltpu.sync_copy(eosbuf, out_hbm.at[pl.ds(ob + E * C, C)])

        @pl.when(h == 0)
        def _():
            pltpu.sync_copy(bosbuf, out_hbm.at[pl.ds(ob, C)])

        @pl.when(wid == 0)
        def _():
            def lbody(bb, vec):
                l0 = row_len(bb)
                return jnp.where(
                    lax.broadcasted_iota(jnp.int32, (B,), 0) == bb, l0 + 2, vec)

            vec = lax.fori_loop(0, B, lbody, jnp.zeros((B,), jnp.int32))
            lenbuf[...] = vec
            pltpu.sync_copy(lenbuf, len_hbm)

    return body(x_flat, mask_i32, bos_emb, eos_emb, zeros)


def kernel(x, bos_emb, eos_emb, padding_mask):
    B, T, C = x.shape
    mask_flat = padding_mask.astype(jnp.int32).reshape(B * T)
    zeros = jnp.zeros((BLK * C,), jnp.float32)
    xe_flat, lengths = _sc_embed(
        x.reshape(B * T * C), mask_flat, bos_emb, eos_emb, zeros, B, T, C)
    xe = xe_flat.reshape(B, T + 2, C)
    new_padding_mask = jnp.arange(T + 2)[None, :] >= lengths[:, None]
    return (xe, new_padding_mask, lengths)
